# baseline (speedup 1.0000x reference)
import jax
import jax.numpy as jnp
import numpy as np
from jax import lax
from jax.experimental import pallas as pl
from jax.experimental.pallas import tpu as pltpu

B, Z_DIM, HID, N_CODES = 16384, 256, 256, 64
BR = 4096

_ROT_A = (13, 15, 26, 6)
_ROT_B = (17, 29, 16, 24)
_KS0 = np.uint32(0)
_KS1 = np.uint32(42)
_KS2 = np.uint32(_KS0 ^ _KS1 ^ np.uint32(0x1BD11BDA))
_TINY = np.float32(np.finfo(np.float32).tiny)


def _rotl(x, r):
    return (x << np.uint32(r)) | (x >> np.uint32(32 - r))


def _tf_rounds(x0, x1, rots):
    for r in rots:
        x0 = (x0 + x1).astype(jnp.uint32)
        x1 = _rotl(x1, r) ^ x0
    return x0, x1


def _threefry_bits(n):
    x0 = (jnp.zeros_like(n) + _KS0).astype(jnp.uint32)
    x1 = (n + _KS1).astype(jnp.uint32)
    for inj, (rots, (a, b, c)) in enumerate((
            (_ROT_A, (_KS1, _KS2, 1)),
            (_ROT_B, (_KS2, _KS0, 2)),
            (_ROT_A, (_KS0, _KS1, 3)),
            (_ROT_B, (_KS1, _KS2, 4)),
            (_ROT_A, (_KS2, _KS0, 5)))):
        x0, x1 = _tf_rounds(x0, x1, rots)
        x0 = (x0 + a).astype(jnp.uint32)
        x1 = (x1 + b + np.uint32(c)).astype(jnp.uint32)
    return x0 ^ x1


def _gumbel_tile(block_idx):
    r = lax.broadcasted_iota(jnp.uint32, (BR // 2, 128), 0)
    c = lax.broadcasted_iota(jnp.uint32, (BR // 2, 128), 1)
    base = (block_idx * (BR * N_CODES)).astype(jnp.uint32)
    n = base + r * np.uint32(N_CODES) + (c & np.uint32(63)) + \
        (c >> np.uint32(6)) * np.uint32(BR // 2 * N_CODES)
    bits = _threefry_bits(n)
    f = lax.bitcast_convert_type(
        (bits >> np.uint32(9)) | np.uint32(0x3F800000), jnp.float32) \
        - np.float32(1.0)
    u = jnp.maximum(_TINY, f * (np.float32(1.0) - _TINY) + _TINY)
    return -jnp.log(-jnp.log(u))


def _mlp_sample_kernel(z_ref, w1_ref, b1_ref, w2_ref, b2_ref, w3_ref, b3_ref,
                       cb_ref, zg_ref, idx_ref, lp_ref):
    g = _gumbel_tile(pl.program_id(0).astype(jnp.uint32))

    z = z_ref[...]
    h1 = jnp.maximum(
        jnp.dot(z, w1_ref[...], preferred_element_type=jnp.float32)
        + b1_ref[...], 0.0)
    h2 = jnp.maximum(
        jnp.dot(h1, w2_ref[...], preferred_element_type=jnp.float32)
        + b2_ref[...], 0.0)
    logits = (jnp.dot(h2, w3_ref[...], preferred_element_type=jnp.float32)
              + b3_ref[...])

    y = logits + jnp.concatenate([g[:, :N_CODES], g[:, N_CODES:]], axis=0)
    iota = lax.broadcasted_iota(jnp.int32, (BR, N_CODES), 1)
    ymax = jnp.max(y, axis=1, keepdims=True)
    idx = jnp.min(jnp.where(y == ymax, iota, N_CODES), axis=1)

    lmax = jnp.max(logits, axis=1, keepdims=True)
    lse = lmax[:, 0] + jnp.log(jnp.sum(jnp.exp(logits - lmax), axis=1))
    onehot = (iota == idx[:, None]).astype(jnp.float32)
    logit_sel = jnp.sum(onehot * logits, axis=1)

    idx_ref[...] = idx
    lp_ref[...] = logit_sel - lse
    zg_ref[...] = jnp.dot(onehot, cb_ref[...],
                          preferred_element_type=jnp.float32)


def kernel(z_cur, w1, b1, w2, b2, w3, b3, codebook):
    b1r, b2r, b3r = (b.reshape(1, -1) for b in (b1, b2, b3))

    full = lambda shape: pl.BlockSpec(shape, lambda i: (0, 0))
    z_goal, code_idx, log_prob = pl.pallas_call(
        _mlp_sample_kernel,
        grid=(B // BR,),
        in_specs=[
            pl.BlockSpec((BR, Z_DIM), lambda i: (i, 0)),
            full((Z_DIM, HID)),
            full((1, HID)),
            full((HID, HID)),
            full((1, HID)),
            full((HID, N_CODES)),
            full((1, N_CODES)),
            full((N_CODES, Z_DIM)),
        ],
        out_specs=[
            pl.BlockSpec((BR, Z_DIM), lambda i: (i, 0)),
            pl.BlockSpec((BR,), lambda i: (i,)),
            pl.BlockSpec((BR,), lambda i: (i,)),
        ],
        out_shape=[
            jax.ShapeDtypeStruct((B, Z_DIM), jnp.float32),
            jax.ShapeDtypeStruct((B,), jnp.int32),
            jax.ShapeDtypeStruct((B,), jnp.float32),
        ],
        compiler_params=pltpu.CompilerParams(
            dimension_semantics=("parallel",)),
    )(z_cur, w1, b1r, w2, b2r, w3, b3r, codebook)
    return (z_goal, code_idx, log_prob)


# speedup vs baseline: 1.0492x; 1.0492x over previous
import jax
import jax.numpy as jnp
import numpy as np
from jax import lax
from jax.experimental import pallas as pl
from jax.experimental.pallas import tpu as pltpu

B, Z_DIM, HID, N_CODES = 16384, 256, 256, 64
BR = 2048

_ROT_A = (13, 15, 26, 6)
_ROT_B = (17, 29, 16, 24)
_KS0 = np.uint32(0)
_KS1 = np.uint32(42)
_KS2 = np.uint32(_KS0 ^ _KS1 ^ np.uint32(0x1BD11BDA))
_TINY = np.float32(np.finfo(np.float32).tiny)


def _rotl(x, r):
    return (x << np.uint32(r)) | (x >> np.uint32(32 - r))


def _tf_rounds(x0, x1, rots):
    for r in rots:
        x0 = (x0 + x1).astype(jnp.uint32)
        x1 = _rotl(x1, r) ^ x0
    return x0, x1


def _threefry_bits(n):
    x1 = (n + _KS1).astype(jnp.uint32)
    x0 = x1
    x1 = _rotl(x1, _ROT_A[0]) ^ x0
    for r in _ROT_A[1:]:
        x0 = (x0 + x1).astype(jnp.uint32)
        x1 = _rotl(x1, r) ^ x0
    x0 = (x0 + _KS1).astype(jnp.uint32)
    x1 = (x1 + _KS2 + np.uint32(1)).astype(jnp.uint32)
    for rots, (a, b, c) in (
            (_ROT_B, (_KS2, _KS0, 2)),
            (_ROT_A, (_KS0, _KS1, 3)),
            (_ROT_B, (_KS1, _KS2, 4)),
            (_ROT_A, (_KS2, _KS0, 5))):
        x0, x1 = _tf_rounds(x0, x1, rots)
        x0 = (x0 + a).astype(jnp.uint32)
        x1 = (x1 + b + np.uint32(c)).astype(jnp.uint32)
    return x0 ^ x1


def _gumbel_tile(block_idx):
    r = lax.broadcasted_iota(jnp.uint32, (BR // 2, 128), 0)
    c = lax.broadcasted_iota(jnp.uint32, (BR // 2, 128), 1)
    base = (block_idx * (BR * N_CODES)).astype(jnp.uint32)
    n = base + r * np.uint32(N_CODES) + (c & np.uint32(63)) + \
        (c >> np.uint32(6)) * np.uint32(BR // 2 * N_CODES)
    bits = _threefry_bits(n)
    f = lax.bitcast_convert_type(
        (bits >> np.uint32(9)) | np.uint32(0x3F800000), jnp.float32) \
        - np.float32(1.0)
    u = jnp.maximum(_TINY, f * (np.float32(1.0) - _TINY) + _TINY)
    return -jnp.log(-jnp.log(u))


def _mlp_sample_kernel(z_ref, w1_ref, b1_ref, w2_ref, b2_ref, w3_ref, b3_ref,
                       cb_ref, zg_ref, idx_ref, lp_ref):
    g = _gumbel_tile(pl.program_id(0).astype(jnp.uint32))

    z = z_ref[...]
    h1 = jnp.maximum(
        jnp.dot(z, w1_ref[...], preferred_element_type=jnp.float32)
        + b1_ref[...], 0.0)
    h2 = jnp.maximum(
        jnp.dot(h1, w2_ref[...], preferred_element_type=jnp.float32)
        + b2_ref[...], 0.0)
    logits = (jnp.dot(h2, w3_ref[...], preferred_element_type=jnp.float32)
              + b3_ref[...])

    y = logits + jnp.concatenate([g[:, :N_CODES], g[:, N_CODES:]], axis=0)
    iota = lax.broadcasted_iota(jnp.int32, (BR, N_CODES), 1)
    ymax = jnp.max(y, axis=1, keepdims=True)
    idx = jnp.min(jnp.where(y == ymax, iota, N_CODES), axis=1)

    onehot = (iota == idx[:, None]).astype(jnp.float32)
    lse = ymax[:, 0] + jnp.log(jnp.sum(jnp.exp(logits - ymax), axis=1))
    logit_sel = jnp.sum(onehot * logits, axis=1)

    idx_ref[...] = idx
    lp_ref[...] = logit_sel - lse
    zg_ref[...] = jnp.dot(onehot, cb_ref[...],
                          preferred_element_type=jnp.float32)


def kernel(z_cur, w1, b1, w2, b2, w3, b3, codebook):
    b1r, b2r, b3r = (b.reshape(1, -1) for b in (b1, b2, b3))

    full = lambda shape: pl.BlockSpec(shape, lambda i: (0, 0))
    z_goal, code_idx, log_prob = pl.pallas_call(
        _mlp_sample_kernel,
        grid=(B // BR,),
        in_specs=[
            pl.BlockSpec((BR, Z_DIM), lambda i: (i, 0)),
            full((Z_DIM, HID)),
            full((1, HID)),
            full((HID, HID)),
            full((1, HID)),
            full((HID, N_CODES)),
            full((1, N_CODES)),
            full((N_CODES, Z_DIM)),
        ],
        out_specs=[
            pl.BlockSpec((BR, Z_DIM), lambda i: (i, 0)),
            pl.BlockSpec((BR,), lambda i: (i,)),
            pl.BlockSpec((BR,), lambda i: (i,)),
        ],
        out_shape=[
            jax.ShapeDtypeStruct((B, Z_DIM), jnp.float32),
            jax.ShapeDtypeStruct((B,), jnp.int32),
            jax.ShapeDtypeStruct((B,), jnp.float32),
        ],
        compiler_params=pltpu.CompilerParams(
            dimension_semantics=("parallel",)),
    )(z_cur, w1, b1r, w2, b2r, w3, b3r, codebook)
    return (z_goal, code_idx, log_prob)
